# Initial kernel scaffold; baseline (speedup 1.0000x reference)
import jax
import jax.numpy as jnp
from jax.experimental import pallas as pl


def kernel(h, vec, Wq, bq, Wk, bk, Wv, bv, Wproj, bproj, Wvec, edge_index):
    raise NotImplementedError("write your pallas kernel here")


# speedup vs baseline: 17.2256x; 17.2256x over previous
import functools

import jax
import jax.numpy as jnp
import numpy as np
from jax import lax
from jax.experimental import pallas as pl
from jax.experimental.pallas import tpu as pltpu
from jax.experimental.pallas import tpu_sc as plsc

NN = 10000
EE = 320000
HID = 128
NH = 4
HD = 32
ROW = 144
SCALE = 1.0 / np.sqrt(HD)

NC = 2
NS = 16
NWK = NC * NS
WIN = 64
NWIN = -(-EE // (NWK * WIN))
EPW = WIN * NWIN
EP = EPW * NWK
NP = 10240
NSL = NP // NS
GRP = WIN // 16
PAD_DST = NN + 8


def _pre_body(h_ref, v384_ref, wq_ref, bq_ref, wk_ref, bk_ref, wv_ref,
              bv_ref, wvec_ref, qt_ref, m0_ref, wt0_ref, wt1_ref, wt2_ref,
              vdot_ref, vec3_ref):
    hh = h_ref[...]
    q = jnp.dot(hh, wq_ref[...], preferred_element_type=jnp.float32) + bq_ref[...]
    k = jnp.dot(hh, wk_ref[...], preferred_element_type=jnp.float32) + bk_ref[...]
    v = jnp.dot(hh, wv_ref[...], preferred_element_type=jnp.float32) + bv_ref[...]
    qt_ref[...] = q
    hv_cat = jnp.concatenate([v[:, 64 * t:64 * t + 32] for t in range(NH)], axis=1)
    s1v_cat = jnp.concatenate([v[:, 64 * t + 32:64 * t + 64] for t in range(NH)], axis=1)
    m0_ref[...] = jnp.concatenate([k, hv_cat], axis=1)
    wt_refs = (wt0_ref, wt1_ref, wt2_ref)
    zpad = jnp.zeros((hh.shape[0], ROW - HID), jnp.float32)
    vdot = None
    for c in range(3):
        vc = v384_ref[:, 128 * c:128 * (c + 1)]
        wt_refs[c][...] = jnp.concatenate([vc * s1v_cat, zpad], axis=1)
        vp = jnp.dot(vc, wvec_ref[...], preferred_element_type=jnp.float32)
        v1 = jnp.concatenate([vp[:, 96 * t:96 * t + 32] for t in range(NH)], axis=1)
        v2 = jnp.concatenate([vp[:, 96 * t + 32:96 * t + 64] for t in range(NH)], axis=1)
        v3 = jnp.concatenate([vp[:, 96 * t + 64:96 * t + 96] for t in range(NH)], axis=1)
        vdot = v1 * v2 if vdot is None else vdot + v1 * v2
        vec3_ref[:, 128 * c:128 * (c + 1)] = v3
    vdot_ref[...] = vdot


def _pre_stage(h, v384, Wq, bq, Wk, bk, Wv, bv, Wvec):
    B = 1000
    grid = (NN // B,)
    row = lambda i: (i, 0)
    full = lambda i: (0, 0)
    return pl.pallas_call(
        _pre_body,
        grid=grid,
        in_specs=[
            pl.BlockSpec((B, HID), row),
            pl.BlockSpec((B, 384), row),
            pl.BlockSpec((HID, HID), full),
            pl.BlockSpec((1, HID), full),
            pl.BlockSpec((HID, HID), full),
            pl.BlockSpec((1, HID), full),
            pl.BlockSpec((HID, 256), full),
            pl.BlockSpec((1, 256), full),
            pl.BlockSpec((HID, 384), full),
        ],
        out_specs=[
            pl.BlockSpec((B, HID), row),
            pl.BlockSpec((B, 256), row),
            pl.BlockSpec((B, ROW), row),
            pl.BlockSpec((B, ROW), row),
            pl.BlockSpec((B, ROW), row),
            pl.BlockSpec((B, HID), row),
            pl.BlockSpec((B, 384), row),
        ],
        out_shape=[
            jax.ShapeDtypeStruct((NN, HID), jnp.float32),
            jax.ShapeDtypeStruct((NN, 256), jnp.float32),
            jax.ShapeDtypeStruct((NN, ROW), jnp.float32),
            jax.ShapeDtypeStruct((NN, ROW), jnp.float32),
            jax.ShapeDtypeStruct((NN, ROW), jnp.float32),
            jax.ShapeDtypeStruct((NN, HID), jnp.float32),
            jax.ShapeDtypeStruct((NN, 384), jnp.float32),
        ],
    )(h, v384, Wq, bq, Wk, bk, Wv, bv, Wvec)


def _sc_body(src_hbm, dst_hbm, qt_hbm, m0_hbm, wt0_hbm, wt1_hbm, wt2_hbm,
             ox_out, vg0_out, vg1_out, vg2_out,
             srcv, dstv, qd, ms, oxw, sptab):
    c = lax.axis_index("c")
    s = lax.axis_index("s")
    w = s * NC + c
    ebase = w * EPW
    rbase = s * NSL
    iota = lax.iota(jnp.int32, 16)
    z16 = jnp.zeros((16,), jnp.float32)

    def _zero_oxw(i, _):
        for j in range(ROW // 16):
            oxw[i, pl.ds(j * 16, 16)] = z16
        return 0

    def _zero_sptab():
        for j in range(10):
            pltpu.sync_copy(oxw, sptab.at[pl.ds(rbase + j * WIN, WIN)])

    lax.fori_loop(0, WIN, _zero_oxw, 0)
    _zero_sptab()
    plsc.subcore_barrier()

    def _g0_win(iw, _):
        b = ebase + iw * WIN
        pltpu.sync_copy(src_hbm.at[pl.ds(b, WIN)], srcv)
        pltpu.sync_copy(dst_hbm.at[pl.ds(b, WIN)], dstv)
        pltpu.sync_copy(qt_hbm.at[dstv], qd)
        pltpu.sync_copy(m0_hbm.at[srcv], ms)
        for g in range(GRP):
            rows = iota + g * 16

            def _dots(dd, accs):
                out = []
                for t in range(NH):
                    col = jnp.full((16,), t * HD, jnp.int32) + dd
                    qv = plsc.load_gather(qd, [rows, col])
                    kv = plsc.load_gather(ms, [rows, col])
                    out.append(accs[t] + qv * kv)
                return tuple(out)

            acc = lax.fori_loop(0, HD, _dots, (z16, z16, z16, z16))
            exs = [jnp.exp(acc[t] * SCALE) for t in range(NH)]
            for t in range(NH):
                plsc.store_scatter(oxw, [rows, jnp.full((16,), HID + t, jnp.int32)], exs[t])

            def _oxb(dd, carry):
                for t in range(NH):
                    col = jnp.full((16,), t * HD, jnp.int32) + dd
                    hv = plsc.load_gather(ms, [rows, col + 128])
                    plsc.store_scatter(oxw, [rows, col], hv * exs[t])
                return carry

            lax.fori_loop(0, HD, _oxb, 0)
        pltpu.sync_copy(oxw, sptab.at[dstv], add=True)
        return 0

    lax.fori_loop(0, NWIN, _g0_win, 0)
    plsc.subcore_barrier()
    for j in range(5):
        pltpu.sync_copy(sptab.at[pl.ds(rbase + j * 128, 128)],
                        ox_out.at[c, pl.ds(rbase + j * 128, 128)])
    plsc.subcore_barrier()

    for wt_tab, vg_out in ((wt0_hbm, vg0_out), (wt1_hbm, vg1_out), (wt2_hbm, vg2_out)):
        lax.fori_loop(0, WIN, _zero_oxw, 0)
        _zero_sptab()
        plsc.subcore_barrier()

        def _gc_win(iw, _, _tab=wt_tab):
            b = ebase + iw * WIN
            pltpu.sync_copy(src_hbm.at[pl.ds(b, WIN)], srcv)
            pltpu.sync_copy(dst_hbm.at[pl.ds(b, WIN)], dstv)
            pltpu.sync_copy(_tab.at[srcv], oxw)
            pltpu.sync_copy(oxw, sptab.at[dstv], add=True)
            return 0

        lax.fori_loop(0, NWIN, _gc_win, 0)
        plsc.subcore_barrier()
        for j in range(5):
            pltpu.sync_copy(sptab.at[pl.ds(rbase + j * 128, 128)],
                            vg_out.at[c, pl.ds(rbase + j * 128, 128)])
        plsc.subcore_barrier()


def _sc_stage(src, dst, qt, m0, wt0, wt1, wt2):
    mesh = plsc.VectorSubcoreMesh(core_axis_name="c", subcore_axis_name="s")
    fn = functools.partial(
        pl.kernel,
        mesh=mesh,
        compiler_params=pltpu.CompilerParams(
            needs_layout_passes=False, use_tc_tiling_on_sc=False),
        out_type=[
            jax.ShapeDtypeStruct((NC, NP, ROW), jnp.float32),
            jax.ShapeDtypeStruct((NC, NP, ROW), jnp.float32),
            jax.ShapeDtypeStruct((NC, NP, ROW), jnp.float32),
            jax.ShapeDtypeStruct((NC, NP, ROW), jnp.float32),
        ],
        scratch_types=[
            pltpu.VMEM((WIN,), jnp.int32),
            pltpu.VMEM((WIN,), jnp.int32),
            pltpu.VMEM((WIN, HID), jnp.float32),
            pltpu.VMEM((WIN, 256), jnp.float32),
            pltpu.VMEM((WIN, ROW), jnp.float32),
            pltpu.VMEM_SHARED((NP, ROW), jnp.float32),
        ],
    )(_sc_body)
    return fn(src, dst, qt, m0, wt0, wt1, wt2)


def _post_body(h_ref, v384_ref, ox_ref, vg0_ref, vg1_ref, vg2_ref,
               vdot_ref, vec3_ref, wpb_ref, bpb_ref, ho_ref, vo_ref):
    oxf = ox_ref[0] + ox_ref[1]
    sel = (lax.broadcasted_iota(jnp.int32, (16, HID), 1) // HD
           == lax.broadcasted_iota(jnp.int32, (16, HID), 0)).astype(jnp.float32)
    den128 = jnp.dot(oxf[:, HID:], sel, preferred_element_type=jnp.float32)
    ox = oxf[:, :HID] / (den128 + 1e-16)
    qs = jnp.dot(ox, wpb_ref[...], preferred_element_type=jnp.float32) + bpb_ref[...]
    q1 = jnp.concatenate([qs[:, 96 * t:96 * t + 32] for t in range(NH)], axis=1)
    q2 = jnp.concatenate([qs[:, 96 * t + 32:96 * t + 64] for t in range(NH)], axis=1)
    q3 = jnp.concatenate([qs[:, 96 * t + 64:96 * t + 96] for t in range(NH)], axis=1)
    ho_ref[...] = h_ref[...] + q2 + q3 * vdot_ref[...]
    vg_refs = (vg0_ref, vg1_ref, vg2_ref)
    pieces = []
    for c in range(3):
        vagg = vg_refs[c][0] + vg_refs[c][1]
        pieces.append(vec3_ref[:, 128 * c:128 * (c + 1)] * q1 + vagg[:, :HID])
    vo_ref[...] = v384_ref[...] + jnp.concatenate(pieces, axis=1)


def _post_stage(h, v384, ox, vg0, vg1, vg2, vdot, vec3, wpb, bpb):
    B = 1000
    grid = (NN // B,)
    row = lambda i: (i, 0)
    prow = lambda i: (0, i, 0)
    full = lambda i: (0, 0)
    return pl.pallas_call(
        _post_body,
        grid=grid,
        in_specs=[
            pl.BlockSpec((B, HID), row),
            pl.BlockSpec((B, 384), row),
            pl.BlockSpec((NC, B, ROW), prow),
            pl.BlockSpec((NC, B, ROW), prow),
            pl.BlockSpec((NC, B, ROW), prow),
            pl.BlockSpec((NC, B, ROW), prow),
            pl.BlockSpec((B, HID), row),
            pl.BlockSpec((B, 384), row),
            pl.BlockSpec((HID, 384), full),
            pl.BlockSpec((1, 384), full),
        ],
        out_specs=[
            pl.BlockSpec((B, HID), row),
            pl.BlockSpec((B, 384), row),
        ],
        out_shape=[
            jax.ShapeDtypeStruct((NN, HID), jnp.float32),
            jax.ShapeDtypeStruct((NN, 384), jnp.float32),
        ],
    )(h, v384, ox, vg0, vg1, vg2, vdot, vec3, wpb, bpb)


def kernel(h, vec, Wq, bq, Wk, bk, Wv, bv, Wproj, bproj, Wvec, edge_index):
    v384 = vec.reshape(NN, 384)
    qt, m0, wt0, wt1, wt2, vdot, vec3 = _pre_stage(
        h, v384, Wq, bq.reshape(1, HID), Wk, bk.reshape(1, HID),
        Wv, bv.reshape(1, 256), Wvec)
    pad = EP - EE
    src = jnp.concatenate([edge_index[0], jnp.zeros((pad,), jnp.int32)])
    dst = jnp.concatenate([edge_index[1], jnp.full((pad,), PAD_DST, jnp.int32)])
    ox, vg0, vg1, vg2 = _sc_stage(src, dst, qt, m0, wt0, wt1, wt2)
    wpb = jnp.zeros((HID, 3 * HID), jnp.float32)
    for t in range(NH):
        wpb = wpb.at[HD * t:HD * (t + 1), 96 * t:96 * (t + 1)].set(Wproj)
    bpb = jnp.tile(bproj, (NH,)).reshape(1, 3 * HID)
    ho, vo384 = _post_stage(h, v384, ox, vg0, vg1, vg2, vdot, vec3, wpb, bpb)
    return ho, vo384.reshape(NN, 3, HID)
